# initial kernel scaffold (unmeasured)
import jax
import jax.numpy as jnp
from jax import lax
from jax.experimental import pallas as pl
from jax.experimental.pallas import tpu as pltpu


def kernel(x, pi):
    x2 = x[0]
    m, n = x2.shape

    def body(x_ref, pi_ref, out_ref, send_buf, send_sem, recv_sem):
        my_x = lax.axis_index("x")
        my_y = lax.axis_index("y")
        dst_y = pi_ref[my_y]

        @pl.when(dst_y != my_y)
        def _swap():
            send_buf[...] = x_ref[...].astype(jnp.bfloat16)
            rdma = pltpu.make_async_remote_copy(
                src_ref=send_buf,
                dst_ref=out_ref,
                send_sem=send_sem,
                recv_sem=recv_sem,
                device_id=(my_x, dst_y),
                device_id_type=pl.DeviceIdType.MESH,
            )
            rdma.start()
            rdma.wait()

        @pl.when(dst_y == my_y)
        def _identity():
            out_ref[...] = x_ref[...].astype(jnp.bfloat16)

    out = pl.pallas_call(
        body,
        out_shape=jax.ShapeDtypeStruct((m, n), jnp.bfloat16),
        in_specs=[
            pl.BlockSpec(memory_space=pltpu.VMEM),
            pl.BlockSpec(memory_space=pltpu.SMEM),
        ],
        out_specs=pl.BlockSpec(memory_space=pltpu.VMEM),
        scratch_shapes=[
            pltpu.VMEM((m, n), jnp.bfloat16),
            pltpu.SemaphoreType.DMA,
            pltpu.SemaphoreType.DMA,
        ],
        compiler_params=pltpu.CompilerParams(collective_id=0),
    )(x2, pi)
    return out[None]


# baseline (device time: 34545 ns/iter reference)
import jax
import jax.numpy as jnp
from jax import lax
from jax.experimental import pallas as pl
from jax.experimental.pallas import tpu as pltpu


def kernel(x, pi):
    x2 = x[0]
    m, n = x2.shape

    def body(x_ref, pi_ref, out_ref, send_buf, send_sem, recv_sem):
        my_x = lax.axis_index("x")
        my_y = lax.axis_index("y")
        dst_y = pi_ref[my_y]

        @pl.when(dst_y != my_y)
        def _swap():
            send_buf[...] = x_ref[...].astype(jnp.bfloat16)
            rdma = pltpu.make_async_remote_copy(
                src_ref=send_buf,
                dst_ref=out_ref,
                send_sem=send_sem,
                recv_sem=recv_sem,
                device_id=(my_x, dst_y),
                device_id_type=pl.DeviceIdType.MESH,
            )
            rdma.start()
            rdma.wait()

        @pl.when(dst_y == my_y)
        def _identity():
            out_ref[...] = x_ref[...].astype(jnp.bfloat16)

    out = pl.pallas_call(
        body,
        out_shape=jax.ShapeDtypeStruct((m, n), jnp.bfloat16),
        in_specs=[
            pl.BlockSpec(memory_space=pltpu.VMEM),
            pl.BlockSpec(memory_space=pltpu.SMEM),
        ],
        out_specs=pl.BlockSpec(memory_space=pltpu.VMEM),
        scratch_shapes=[
            pltpu.VMEM((m, n), jnp.bfloat16),
            pltpu.SemaphoreType.DMA,
            pltpu.SemaphoreType.DMA,
        ],
    )(x2, pi)
    return out[None]


# device time: 26841 ns/iter; 1.2870x vs baseline; 1.2870x over previous
import jax
import jax.numpy as jnp
from jax import lax
from jax.experimental import pallas as pl
from jax.experimental.pallas import tpu as pltpu

NC = 8


def kernel(x, pi):
    x2 = x[0]
    m, n = x2.shape
    half = m // 2
    ch = half // NC

    def body(x_ref, pi_ref, out_ref, stage,
             send1, recv1, send2, recv2):
        my_x = lax.axis_index("x")
        my_y = lax.axis_index("y")
        dst_y = pi_ref[my_y]

        @pl.when(dst_y != my_y)
        def _swap():
            stage[...] = x_ref[...].astype(jnp.bfloat16)

            my_rows = my_x * half
            peer_rows = (1 - my_x) * half

            rdma1 = []
            for c in range(NC):
                r = my_rows + c * ch
                d = pltpu.make_async_remote_copy(
                    src_ref=stage.at[pl.ds(r, ch), :],
                    dst_ref=out_ref.at[pl.ds(r, ch), :],
                    send_sem=send1.at[c],
                    recv_sem=recv1.at[c],
                    device_id=(my_x, dst_y),
                    device_id_type=pl.DeviceIdType.MESH,
                )
                d.start()
                rdma1.append(d)

            rdma2 = []
            for c in range(NC):
                rdma1[c].wait_recv()
                r = my_rows + c * ch
                d = pltpu.make_async_remote_copy(
                    src_ref=out_ref.at[pl.ds(r, ch), :],
                    dst_ref=out_ref.at[pl.ds(r, ch), :],
                    send_sem=send2.at[c],
                    recv_sem=recv2.at[c],
                    device_id=(1 - my_x, my_y),
                    device_id_type=pl.DeviceIdType.MESH,
                )
                d.start()
                rdma2.append(d)

            for c in range(NC):
                rdma2[c].wait_recv()
            for c in range(NC):
                rdma1[c].wait_send()
                rdma2[c].wait_send()

        @pl.when(dst_y == my_y)
        def _identity():
            out_ref[...] = x_ref[...].astype(jnp.bfloat16)

    out = pl.pallas_call(
        body,
        out_shape=jax.ShapeDtypeStruct((m, n), jnp.bfloat16),
        in_specs=[
            pl.BlockSpec(memory_space=pltpu.VMEM),
            pl.BlockSpec(memory_space=pltpu.SMEM),
        ],
        out_specs=pl.BlockSpec(memory_space=pltpu.VMEM),
        scratch_shapes=[
            pltpu.VMEM((m, n), jnp.bfloat16),
            pltpu.SemaphoreType.DMA((NC,)),
            pltpu.SemaphoreType.DMA((NC,)),
            pltpu.SemaphoreType.DMA((NC,)),
            pltpu.SemaphoreType.DMA((NC,)),
        ],
    )(x2, pi)
    return out[None]


# device time: 23438 ns/iter; 1.4739x vs baseline; 1.1452x over previous
import jax
import jax.numpy as jnp
from jax import lax
from jax.experimental import pallas as pl
from jax.experimental.pallas import tpu as pltpu

NC = 8


def kernel(x, pi):
    x2 = x[0]
    m, n = x2.shape
    half = m // 2
    ch = half // NC

    def body(x_ref, pi_ref, out_ref, stage,
             send1, recv1, send2, recv2):
        my_x = lax.axis_index("x")
        my_y = lax.axis_index("y")
        dst_y = pi_ref[my_y]

        @pl.when(dst_y != my_y)
        def _swap():
            barrier = pltpu.get_barrier_semaphore()
            for nbr in ((my_x, 1 - my_y), (1 - my_x, my_y)):
                pl.semaphore_signal(
                    barrier, inc=1,
                    device_id=nbr, device_id_type=pl.DeviceIdType.MESH,
                )
            pl.semaphore_wait(barrier, 2)

            for mx in (0, 1):
                @pl.when(my_x == mx)
                def _(mx=mx):
                    base = mx * half

                    rdma1 = []
                    for c in range(NC):
                        r = base + c * ch
                        stage[r:r + ch, :] = (
                            x_ref[r:r + ch, :].astype(jnp.bfloat16)
                        )
                        d = pltpu.make_async_remote_copy(
                            src_ref=stage.at[pl.ds(r, ch), :],
                            dst_ref=out_ref.at[pl.ds(r, ch), :],
                            send_sem=send1.at[c],
                            recv_sem=recv1.at[c],
                            device_id=(mx, dst_y),
                            device_id_type=pl.DeviceIdType.MESH,
                        )
                        d.start()
                        rdma1.append(d)

                    rdma2 = []
                    for c in range(NC):
                        rdma1[c].wait_recv()
                        r = base + c * ch
                        d = pltpu.make_async_remote_copy(
                            src_ref=out_ref.at[pl.ds(r, ch), :],
                            dst_ref=out_ref.at[pl.ds(r, ch), :],
                            send_sem=send2.at[c],
                            recv_sem=recv2.at[c],
                            device_id=(1 - mx, my_y),
                            device_id_type=pl.DeviceIdType.MESH,
                        )
                        d.start()
                        rdma2.append(d)

                    for c in range(NC):
                        rdma2[c].wait_recv()
                    for c in range(NC):
                        rdma1[c].wait_send()
                        rdma2[c].wait_send()

        @pl.when(dst_y == my_y)
        def _identity():
            out_ref[...] = x_ref[...].astype(jnp.bfloat16)

    out = pl.pallas_call(
        body,
        out_shape=jax.ShapeDtypeStruct((m, n), jnp.bfloat16),
        in_specs=[
            pl.BlockSpec(memory_space=pltpu.VMEM),
            pl.BlockSpec(memory_space=pltpu.SMEM),
        ],
        out_specs=pl.BlockSpec(memory_space=pltpu.VMEM),
        scratch_shapes=[
            pltpu.VMEM((m, n), jnp.bfloat16),
            pltpu.SemaphoreType.DMA((NC,)),
            pltpu.SemaphoreType.DMA((NC,)),
            pltpu.SemaphoreType.DMA((NC,)),
            pltpu.SemaphoreType.DMA((NC,)),
        ],
        compiler_params=pltpu.CompilerParams(collective_id=0),
    )(x2, pi)
    return out[None]
